# baseline (speedup 1.0000x reference)
import functools

import jax
import jax.numpy as jnp
from jax import lax
from jax.experimental import pallas as pl
from jax.experimental.pallas import tpu as pltpu
from jax.experimental.pallas import tpu_sc as plsc

EMBED = 64
LANES = 16
CHUNK = 512
SUB = 128


def _rsqrt_nr(s):
    i = lax.bitcast_convert_type(s, jnp.int32)
    i = jnp.int32(0x5F3759DF) - lax.shift_right_arithmetic(i, jnp.int32(1))
    y = lax.bitcast_convert_type(i, jnp.float32)
    for _ in range(3):
        y = y * (1.5 - 0.5 * s * y * y)
    return y


def _normalize_rows(rows_v, n_groups):
    iota = lax.iota(jnp.int32, LANES)

    def group_body(g, carry):
        ridx = g * LANES + iota
        accs = [jnp.zeros((LANES,), jnp.float32) for _ in range(4)]
        for j in range(EMBED):
            cj = jnp.full((LANES,), j, dtype=jnp.int32)
            c = plsc.load_gather(rows_v, [ridx, cj])
            accs[j % 4] = accs[j % 4] + c * c
        s = (accs[0] + accs[1]) + (accs[2] + accs[3])
        y = _rsqrt_nr(s)
        d = s * y + 1e-8
        z = _rsqrt_nr(d)
        inv = z * z
        for j in range(EMBED):
            cj = jnp.full((LANES,), j, dtype=jnp.int32)
            c = plsc.load_gather(rows_v, [ridx, cj])
            plsc.store_scatter(rows_v, [ridx, cj], c * inv)
        return carry

    lax.fori_loop(0, n_groups, group_body, 0)


@functools.cache
def _build(B):
    info = plsc.get_sparse_core_info()
    nc, ns = info.num_cores, info.num_subcores
    nw = nc * ns
    per_w = B // nw
    n_chunks = per_w // CHUNK
    idx_rows_w = per_w // SUB
    subs = CHUNK // SUB

    mesh = plsc.VectorSubcoreMesh(core_axis_name="c", subcore_axis_name="s")

    @functools.partial(
        pl.kernel,
        mesh=mesh,
        compiler_params=pltpu.CompilerParams(
            needs_layout_passes=False, use_tc_tiling_on_sc=False),
        out_type=jax.ShapeDtypeStruct((B, EMBED), jnp.float32),
        scratch_types=[
            pltpu.VMEM((idx_rows_w, SUB), jnp.int32),
            pltpu.VMEM((CHUNK, EMBED), jnp.float32),
            pltpu.SemaphoreType.DMA,
        ],
    )
    def sc_fn(matrix_hbm, tok_hbm, out_hbm, idx_v, rows_v, sem):
        wid = lax.axis_index("s") * nc + lax.axis_index("c")
        pltpu.sync_copy(tok_hbm.at[pl.ds(wid * idx_rows_w, idx_rows_w)], idx_v)

        def chunk_body(t, carry):
            base = wid * per_w + t * CHUNK
            handles = []
            for k in range(subs):
                handles.append(pltpu.async_copy(
                    matrix_hbm.at[idx_v.at[t * subs + k]],
                    rows_v.at[pl.ds(k * SUB, SUB)],
                    sem))
            for h in handles:
                h.wait()
            _normalize_rows(rows_v, CHUNK // LANES)
            pltpu.sync_copy(rows_v, out_hbm.at[pl.ds(base, CHUNK)])
            return carry

        lax.fori_loop(0, n_chunks, chunk_body, 0)

    return sc_fn


def kernel(matrix, tokens):
    nb, nt = tokens.shape
    b = nb * nt
    tok = tokens.reshape(-1).astype(jnp.int32).reshape(-1, SUB)
    out = _build(b)(matrix, tok)
    return out.reshape(nb, nt, EMBED)


# speedup vs baseline: 2.4037x; 2.4037x over previous
import functools

import jax
import jax.numpy as jnp
from jax import lax
from jax.experimental import pallas as pl
from jax.experimental.pallas import tpu as pltpu
from jax.experimental.pallas import tpu_sc as plsc

EMBED = 64
LANES = 16
CHUNK = 512
SUB = 128


def _rsqrt_nr(s):
    i = lax.bitcast_convert_type(s, jnp.int32)
    i = jnp.int32(0x5F3759DF) - lax.shift_right_arithmetic(i, jnp.int32(1))
    y = lax.bitcast_convert_type(i, jnp.float32)
    for _ in range(3):
        y = y * (1.5 - 0.5 * s * y * y)
    return y


GROUP = 16


def _normalize_rows(rows_v, n_groups):
    nparts = EMBED // LANES

    def group_body(g, carry):
        base = g * GROUP
        parts = []
        invs = []
        for rr in range(GROUP):
            p = [rows_v[base + rr, pl.ds(c * LANES, LANES)]
                 for c in range(nparts)]
            parts.append(p)
            sv = (p[0] * p[0] + p[1] * p[1]) + (p[2] * p[2] + p[3] * p[3])
            s = jnp.sum(sv)
            y = _rsqrt_nr(s)
            d = s * y + 1e-8
            z = _rsqrt_nr(d)
            invs.append(z * z)
        for rr in range(GROUP):
            for c in range(nparts):
                rows_v[base + rr, pl.ds(c * LANES, LANES)] = (
                    parts[rr][c] * invs[rr])
        return carry

    lax.fori_loop(0, n_groups, group_body, 0)


@functools.cache
def _build(B):
    info = plsc.get_sparse_core_info()
    nc, ns = info.num_cores, info.num_subcores
    nw = nc * ns
    per_w = B // nw
    n_chunks = per_w // CHUNK
    idx_rows_w = per_w // SUB
    subs = CHUNK // SUB

    mesh = plsc.VectorSubcoreMesh(core_axis_name="c", subcore_axis_name="s")

    @functools.partial(
        pl.kernel,
        mesh=mesh,
        compiler_params=pltpu.CompilerParams(
            needs_layout_passes=False, use_tc_tiling_on_sc=False),
        out_type=jax.ShapeDtypeStruct((B, EMBED), jnp.float32),
        scratch_types=[
            pltpu.VMEM((idx_rows_w, SUB), jnp.int32),
            pltpu.VMEM((CHUNK, EMBED), jnp.float32),
            pltpu.SemaphoreType.DMA,
        ],
    )
    def sc_fn(matrix_hbm, tok_hbm, out_hbm, idx_v, rows_v, sem):
        wid = lax.axis_index("s") * nc + lax.axis_index("c")
        pltpu.sync_copy(tok_hbm.at[pl.ds(wid * idx_rows_w, idx_rows_w)], idx_v)

        def chunk_body(t, carry):
            base = wid * per_w + t * CHUNK
            handles = []
            for k in range(subs):
                handles.append(pltpu.async_copy(
                    matrix_hbm.at[idx_v.at[t * subs + k]],
                    rows_v.at[pl.ds(k * SUB, SUB)],
                    sem))
            for h in handles:
                h.wait()
            _normalize_rows(rows_v, CHUNK // LANES)
            pltpu.sync_copy(rows_v, out_hbm.at[pl.ds(base, CHUNK)])
            return carry

        lax.fori_loop(0, n_chunks, chunk_body, 0)

    return sc_fn


def kernel(matrix, tokens):
    nb, nt = tokens.shape
    b = nb * nt
    tok = tokens.reshape(-1).astype(jnp.int32).reshape(-1, SUB)
    out = _build(b)(matrix, tok)
    return out.reshape(nb, nt, EMBED)
